# baseline (speedup 1.0000x reference)
import jax
import jax.numpy as jnp
from jax import lax
from jax.experimental import pallas as pl
from jax.experimental.pallas import tpu as pltpu
from jax.experimental.pallas import tpu_sc as plsc

BATCH = 16384
EMBED_DIM = 64
NUM_FEATURES = 13
NFB = NUM_FEATURES + 1
NC = 2
NS = 16
NW = NC * NS
CHUNK = BATCH // NW
IDX_SUB = 64
NSUB = CHUNK // IDX_SUB
PASSES = 2
SPP = NSUB // PASSES
EPP = CHUNK // PASSES
GPS = IDX_SUB // 16


def _sc_body(uprow_hbm, mprow_hbm, upar_hbm, mpar_hbm, feat_hbm,
             utab_hbm, mtab_hbm, wbc_hbm, out_hbm,
             uprow_v, mprow_v, upar_v, mpar_v, feat_v, wbc_v,
             upair_v, mpair_v, out_v, sem_u, sem_m):
    cid = lax.axis_index("c")
    sid = lax.axis_index("s")
    wid = sid * NC + cid
    base = wid * CHUNK

    pltpu.sync_copy(uprow_hbm.at[wid], uprow_v)
    pltpu.sync_copy(mprow_hbm.at[wid], mprow_v)
    pltpu.sync_copy(upar_hbm.at[pl.ds(base, CHUNK)], upar_v)
    pltpu.sync_copy(mpar_hbm.at[pl.ds(base, CHUNK)], mpar_v)
    pltpu.sync_copy(feat_hbm.at[wid], feat_v)
    pltpu.sync_copy(wbc_hbm, wbc_v)

    lane = lax.iota(jnp.int32, 16)

    def make_group(p):
        def group(g, carry):
            eloc = p * EPP + g * 16
            rloc = g * 16 + lane
            cu = upar_v[pl.ds(eloc, 16)]
            cm = mpar_v[pl.ds(eloc, 16)]
            accs = [jnp.zeros((16,), jnp.float32) for _ in range(4)]
            for d in range(EMBED_DIM):
                u = plsc.load_gather(upair_v, [rloc, cu])
                m = plsc.load_gather(mpair_v, [rloc, cm])
                wv = wbc_v[d // 8, pl.ds((d % 8) * 16, 16)]
                accs[d % 4] = accs[d % 4] + (u * m) * wv
                cu = cu + 1
                cm = cm + 1
            frows = (eloc + lane) >> 3
            cf = ((eloc + lane) & 7) << 4
            for f in range(NFB):
                fv = plsc.load_gather(feat_v, [frows, cf])
                j = EMBED_DIM + f
                accs[f % 4] = accs[f % 4] + fv * wbc_v[
                    j // 8, pl.ds((j % 8) * 16, 16)]
                cf = cf + 1
            out_v[pl.ds(eloc, 16)] = (accs[0] + accs[1]) + (accs[2] + accs[3])
            return carry
        return group

    for p in range(PASSES):
        copies = []
        for j in range(SPP):
            s = p * SPP + j
            copies.append(pltpu.async_copy(
                utab_hbm.at[uprow_v.at[s]],
                upair_v.at[pl.ds(j * IDX_SUB, IDX_SUB)], sem_u))
            copies.append(pltpu.async_copy(
                mtab_hbm.at[mprow_v.at[s]],
                mpair_v.at[pl.ds(j * IDX_SUB, IDX_SUB)], sem_m))
        group = make_group(p)
        for j in range(SPP):
            copies[2 * j].wait()
            copies[2 * j + 1].wait()
            lax.fori_loop(j * GPS, (j + 1) * GPS, group, None)
    pltpu.sync_copy(out_v, out_hbm.at[pl.ds(base, CHUNK)])


def _transpose_block(xt_ref, eye_ref, out_ref):
    x = lax.dot_general(xt_ref[...], eye_ref[...], (((0,), (0,)), ((), ())),
                        precision=lax.Precision.HIGHEST)
    x3 = x.reshape(TB // 2, 2, EMBED_DIM)
    out_ref[...] = jnp.concatenate([x3[:, 0, :], x3[:, 1, :]], axis=1)


TB = 1024


def _detranspose(tabT):
    n = tabT.shape[1]
    grid = (n + TB - 1) // TB
    eye = jnp.eye(EMBED_DIM, dtype=jnp.float32)
    return pl.pallas_call(
        _transpose_block,
        grid=(grid,),
        in_specs=[pl.BlockSpec((EMBED_DIM, TB), lambda i: (0, i)),
                  pl.BlockSpec((EMBED_DIM, EMBED_DIM), lambda i: (0, 0))],
        out_specs=pl.BlockSpec((TB // 2, 128), lambda i: (i, 0)),
        out_shape=jax.ShapeDtypeStruct((n // 2, 128), jnp.float32),
    )(tabT, eye)


@jax.jit
def _run(uprow, mprow, upar, mpar, feat16, utab2, mtab2, wbc):
    mesh = plsc.VectorSubcoreMesh(core_axis_name="c", subcore_axis_name="s",
                                  num_cores=NC, num_subcores=NS)
    f = pl.kernel(
        _sc_body,
        out_type=jax.ShapeDtypeStruct((BATCH,), jnp.float32),
        mesh=mesh,
        compiler_params=pltpu.CompilerParams(needs_layout_passes=False),
        scratch_types=[
            pltpu.VMEM((NSUB, IDX_SUB), jnp.int32),
            pltpu.VMEM((NSUB, IDX_SUB), jnp.int32),
            pltpu.VMEM((CHUNK,), jnp.int32),
            pltpu.VMEM((CHUNK,), jnp.int32),
            pltpu.VMEM((CHUNK // 8, 128), jnp.float32),
            pltpu.VMEM((10, 128), jnp.float32),
            pltpu.VMEM((EPP, 128), jnp.float32),
            pltpu.VMEM((EPP, 128), jnp.float32),
            pltpu.VMEM((CHUNK,), jnp.float32),
            pltpu.SemaphoreType.DMA,
            pltpu.SemaphoreType.DMA,
        ],
    )
    return f(uprow, mprow, upar, mpar, feat16, utab2, mtab2, wbc)


def kernel(user_idx, movie_idx, features, user_table, movie_table, fc_w, fc_b):
    ui = user_idx.astype(jnp.int32)
    mi = movie_idx.astype(jnp.int32)
    uprow = (ui // 2).reshape(NW, NSUB, IDX_SUB)
    mprow = (mi // 2).reshape(NW, NSUB, IDX_SUB)
    upar = (ui % 2) * 64
    mpar = (mi % 2) * 64
    feat16 = jnp.concatenate(
        [features,
         jnp.ones((BATCH, 1), jnp.float32),
         jnp.zeros((BATCH, 2), jnp.float32)], axis=1).reshape(NW, 64, 128)
    params = jnp.concatenate(
        [fc_w[0], fc_b, jnp.zeros((2,), jnp.float32)]).astype(jnp.float32)
    wbc = jnp.tile(params[:, None], (1, 16)).reshape(10, 128)
    utab2 = _detranspose(user_table.T)
    mtab2 = _detranspose(movie_table.T)
    return _run(uprow, mprow, upar, mpar, feat16, utab2, mtab2, wbc)


# speedup vs baseline: 1.9418x; 1.9418x over previous
import jax
import jax.numpy as jnp
from jax import lax
from jax.experimental import pallas as pl
from jax.experimental.pallas import tpu as pltpu
from jax.experimental.pallas import tpu_sc as plsc

BATCH = 16384
EMBED_DIM = 64
NUM_FEATURES = 13
NFB = NUM_FEATURES + 1
NC = 2
NS = 16
NW = NC * NS
CHUNK = BATCH // NW
IDX_SUB = 64
NSUB = CHUNK // IDX_SUB
PASSES = 2
SPP = NSUB // PASSES
EPP = CHUNK // PASSES
GPS = IDX_SUB // 16


def _sc_body(uprow_hbm, mprow_hbm, upar_hbm, mpar_hbm, feat_hbm,
             utab_hbm, mtab_hbm, wbc_hbm, out_hbm,
             uprow_v, mprow_v, upar_v, mpar_v, feat_v, wbc_v,
             upair_v, mpair_v, out_v, sem_u, sem_m):
    cid = lax.axis_index("c")
    sid = lax.axis_index("s")
    wid = sid * NC + cid
    base = wid * CHUNK

    pltpu.sync_copy(uprow_hbm.at[wid], uprow_v)
    pltpu.sync_copy(mprow_hbm.at[wid], mprow_v)
    pltpu.sync_copy(upar_hbm.at[pl.ds(base, CHUNK)], upar_v)
    pltpu.sync_copy(mpar_hbm.at[pl.ds(base, CHUNK)], mpar_v)
    pltpu.sync_copy(feat_hbm.at[wid], feat_v)
    pltpu.sync_copy(wbc_hbm, wbc_v)

    lane = lax.iota(jnp.int32, 16)

    def make_group(p):
        def group(g, carry):
            eloc = p * EPP + g * 16
            rloc = g * 16 + lane
            cu = upar_v[pl.ds(eloc, 16)]
            cm = mpar_v[pl.ds(eloc, 16)]
            accs = [jnp.zeros((16,), jnp.float32) for _ in range(4)]
            for d in range(EMBED_DIM):
                u = plsc.load_gather(upair_v, [rloc, cu])
                m = plsc.load_gather(mpair_v, [rloc, cm])
                wv = wbc_v[d // 8, pl.ds((d % 8) * 16, 16)]
                accs[d % 4] = accs[d % 4] + (u * m) * wv
                cu = cu + 1
                cm = cm + 1
            frows = (eloc + lane) >> 3
            cf = ((eloc + lane) & 7) << 4
            for f in range(NFB):
                fv = plsc.load_gather(feat_v, [frows, cf])
                j = EMBED_DIM + f
                accs[f % 4] = accs[f % 4] + fv * wbc_v[
                    j // 8, pl.ds((j % 8) * 16, 16)]
                cf = cf + 1
            out_v[pl.ds(eloc, 16)] = (accs[0] + accs[1]) + (accs[2] + accs[3])
            return carry
        return group

    for p in range(PASSES):
        copies = []
        for j in range(SPP):
            s = p * SPP + j
            copies.append(pltpu.async_copy(
                utab_hbm.at[uprow_v.at[s]],
                upair_v.at[pl.ds(j * IDX_SUB, IDX_SUB)], sem_u))
            copies.append(pltpu.async_copy(
                mtab_hbm.at[mprow_v.at[s]],
                mpair_v.at[pl.ds(j * IDX_SUB, IDX_SUB)], sem_m))
        group = make_group(p)
        for j in range(SPP):
            copies[2 * j].wait()
            copies[2 * j + 1].wait()
            lax.fori_loop(j * GPS, (j + 1) * GPS, group, None)
    pltpu.sync_copy(out_v, out_hbm.at[pl.ds(base, CHUNK)])


TB = 4096
NROW = 1000000
NBLK = (NROW + TB - 1) // TB
PAIR_ROWS = NBLK * (TB // 2)


def _transpose_block(xt_ref, eye_ref, out_ref):
    x = lax.dot_general(xt_ref[...], eye_ref[...], (((0,), (0,)), ((), ())),
                        precision=lax.Precision.HIGHEST)
    out_ref[:, 0:EMBED_DIM] = x[0:TB // 2, :]
    out_ref[:, EMBED_DIM:128] = x[TB // 2:TB, :]


def _detranspose(tabT):
    eye = jnp.eye(EMBED_DIM, dtype=jnp.float32)
    return pl.pallas_call(
        _transpose_block,
        grid=(NBLK,),
        in_specs=[pl.BlockSpec((EMBED_DIM, TB), lambda i: (0, i)),
                  pl.BlockSpec((EMBED_DIM, EMBED_DIM), lambda i: (0, 0))],
        out_specs=pl.BlockSpec((TB // 2, 128), lambda i: (i, 0)),
        out_shape=jax.ShapeDtypeStruct((PAIR_ROWS, 128), jnp.float32),
    )(tabT, eye)


@jax.jit
def _run(uprow, mprow, upar, mpar, feat16, utab2, mtab2, wbc):
    mesh = plsc.VectorSubcoreMesh(core_axis_name="c", subcore_axis_name="s",
                                  num_cores=NC, num_subcores=NS)
    f = pl.kernel(
        _sc_body,
        out_type=jax.ShapeDtypeStruct((BATCH,), jnp.float32),
        mesh=mesh,
        compiler_params=pltpu.CompilerParams(needs_layout_passes=False),
        scratch_types=[
            pltpu.VMEM((NSUB, IDX_SUB), jnp.int32),
            pltpu.VMEM((NSUB, IDX_SUB), jnp.int32),
            pltpu.VMEM((CHUNK,), jnp.int32),
            pltpu.VMEM((CHUNK,), jnp.int32),
            pltpu.VMEM((CHUNK // 8, 128), jnp.float32),
            pltpu.VMEM((10, 128), jnp.float32),
            pltpu.VMEM((EPP, 128), jnp.float32),
            pltpu.VMEM((EPP, 128), jnp.float32),
            pltpu.VMEM((CHUNK,), jnp.float32),
            pltpu.SemaphoreType.DMA,
            pltpu.SemaphoreType.DMA,
        ],
    )
    return f(uprow, mprow, upar, mpar, feat16, utab2, mtab2, wbc)


def kernel(user_idx, movie_idx, features, user_table, movie_table, fc_w, fc_b):
    ui = user_idx.astype(jnp.int32)
    mi = movie_idx.astype(jnp.int32)
    hb = TB // 2
    uprow = ((ui // TB) * hb + ui % hb).reshape(NW, NSUB, IDX_SUB)
    mprow = ((mi // TB) * hb + mi % hb).reshape(NW, NSUB, IDX_SUB)
    upar = ((ui % TB) // hb) * 64
    mpar = ((mi % TB) // hb) * 64
    feat16 = jnp.concatenate(
        [features,
         jnp.ones((BATCH, 1), jnp.float32),
         jnp.zeros((BATCH, 2), jnp.float32)], axis=1).reshape(NW, 64, 128)
    params = jnp.concatenate(
        [fc_w[0], fc_b, jnp.zeros((2,), jnp.float32)]).astype(jnp.float32)
    wbc = jnp.tile(params[:, None], (1, 16)).reshape(10, 128)
    utab2 = _detranspose(user_table.T)
    mtab2 = _detranspose(movie_table.T)
    return _run(uprow, mprow, upar, mpar, feat16, utab2, mtab2, wbc)


# speedup vs baseline: 3.4153x; 1.7589x over previous
import jax
import jax.numpy as jnp
from jax import lax
from jax.experimental import pallas as pl
from jax.experimental.pallas import tpu as pltpu
from jax.experimental.pallas import tpu_sc as plsc

BATCH = 16384
EMBED_DIM = 64
NUM_FEATURES = 13
NFB = NUM_FEATURES + 1
NC = 2
NS = 16
NW = NC * NS
CHUNK = BATCH // NW
IDX_SUB = 64
NSUB = CHUNK // IDX_SUB
PASSES = 2
SPP = NSUB // PASSES
EPP = CHUNK // PASSES
GPS = IDX_SUB // 16


def _sc_body(uprow_hbm, mprow_hbm, upar_hbm, mpar_hbm, feat_hbm,
             utab_hbm, mtab_hbm, wbc_hbm, out_hbm,
             uprow_v, mprow_v, upar_v, mpar_v, feat_v, wbc_v,
             upair_v, mpair_v, out_v, sem_u, sem_m):
    cid = lax.axis_index("c")
    sid = lax.axis_index("s")
    wid = sid * NC + cid
    base = wid * CHUNK

    pltpu.sync_copy(uprow_hbm.at[wid], uprow_v)
    pltpu.sync_copy(mprow_hbm.at[wid], mprow_v)
    pltpu.sync_copy(upar_hbm.at[pl.ds(base, CHUNK)], upar_v)
    pltpu.sync_copy(mpar_hbm.at[pl.ds(base, CHUNK)], mpar_v)
    pltpu.sync_copy(feat_hbm.at[wid], feat_v)
    pltpu.sync_copy(wbc_hbm, wbc_v)

    lane = lax.iota(jnp.int32, 16)

    def make_group(p):
        def group(g, carry):
            eloc = p * EPP + g * 16
            rloc = g * 16 + lane
            cu = upar_v[pl.ds(eloc, 16)]
            cm = mpar_v[pl.ds(eloc, 16)]
            accs = [jnp.zeros((16,), jnp.float32) for _ in range(4)]
            for d in range(EMBED_DIM):
                u = plsc.load_gather(upair_v, [rloc, cu])
                m = plsc.load_gather(mpair_v, [rloc, cm])
                wv = wbc_v[d // 8, pl.ds((d % 8) * 16, 16)]
                accs[d % 4] = accs[d % 4] + (u * m) * wv
                cu = cu + 1
                cm = cm + 1
            frows = (eloc + lane) >> 3
            cf = ((eloc + lane) & 7) << 4
            for f in range(NFB):
                fv = plsc.load_gather(feat_v, [frows, cf])
                j = EMBED_DIM + f
                accs[f % 4] = accs[f % 4] + fv * wbc_v[
                    j // 8, pl.ds((j % 8) * 16, 16)]
                cf = cf + 1
            out_v[pl.ds(eloc, 16)] = (accs[0] + accs[1]) + (accs[2] + accs[3])
            return carry
        return group

    for p in range(PASSES):
        copies = []
        for j in range(SPP):
            s = p * SPP + j
            copies.append(pltpu.async_copy(
                utab_hbm.at[uprow_v.at[s]],
                upair_v.at[pl.ds(j * IDX_SUB, IDX_SUB)], sem_u))
            copies.append(pltpu.async_copy(
                mtab_hbm.at[mprow_v.at[s]],
                mpair_v.at[pl.ds(j * IDX_SUB, IDX_SUB)], sem_m))
        group = make_group(p)
        for j in range(SPP):
            copies[2 * j].wait()
            copies[2 * j + 1].wait()
            lax.fori_loop(j * GPS, (j + 1) * GPS, group, None)
    pltpu.sync_copy(out_v, out_hbm.at[pl.ds(base, CHUNK)])


TB = 8192
NROW = 1000000
NBLK = (NROW + TB - 1) // TB
PAIR_ROWS = NBLK * (TB // 2)


def _transpose_block(xt_ref, eye_ref, out_ref):
    x = lax.dot_general(xt_ref[...], eye_ref[...], (((0,), (0,)), ((), ())),
                        precision=lax.Precision.DEFAULT)
    out_ref[:, 0:EMBED_DIM] = x[0:TB // 2, :]
    out_ref[:, EMBED_DIM:128] = x[TB // 2:TB, :]


def _detranspose(tabT):
    eye = jnp.eye(EMBED_DIM, dtype=jnp.float32)
    return pl.pallas_call(
        _transpose_block,
        grid=(NBLK,),
        in_specs=[pl.BlockSpec((EMBED_DIM, TB), lambda i: (0, i)),
                  pl.BlockSpec((EMBED_DIM, EMBED_DIM), lambda i: (0, 0))],
        out_specs=pl.BlockSpec((TB // 2, 128), lambda i: (i, 0)),
        out_shape=jax.ShapeDtypeStruct((PAIR_ROWS, 128), jnp.float32),
    )(tabT, eye)


@jax.jit
def _run(uprow, mprow, upar, mpar, feat16, utab2, mtab2, wbc):
    mesh = plsc.VectorSubcoreMesh(core_axis_name="c", subcore_axis_name="s",
                                  num_cores=NC, num_subcores=NS)
    f = pl.kernel(
        _sc_body,
        out_type=jax.ShapeDtypeStruct((BATCH,), jnp.float32),
        mesh=mesh,
        compiler_params=pltpu.CompilerParams(needs_layout_passes=False),
        scratch_types=[
            pltpu.VMEM((NSUB, IDX_SUB), jnp.int32),
            pltpu.VMEM((NSUB, IDX_SUB), jnp.int32),
            pltpu.VMEM((CHUNK,), jnp.int32),
            pltpu.VMEM((CHUNK,), jnp.int32),
            pltpu.VMEM((CHUNK // 8, 128), jnp.float32),
            pltpu.VMEM((10, 128), jnp.float32),
            pltpu.VMEM((EPP, 128), jnp.float32),
            pltpu.VMEM((EPP, 128), jnp.float32),
            pltpu.VMEM((CHUNK,), jnp.float32),
            pltpu.SemaphoreType.DMA,
            pltpu.SemaphoreType.DMA,
        ],
    )
    return f(uprow, mprow, upar, mpar, feat16, utab2, mtab2, wbc)


def kernel(user_idx, movie_idx, features, user_table, movie_table, fc_w, fc_b):
    ui = user_idx.astype(jnp.int32)
    mi = movie_idx.astype(jnp.int32)
    hb = TB // 2
    uprow = ((ui // TB) * hb + ui % hb).reshape(NW, NSUB, IDX_SUB)
    mprow = ((mi // TB) * hb + mi % hb).reshape(NW, NSUB, IDX_SUB)
    upar = ((ui % TB) // hb) * 64
    mpar = ((mi % TB) // hb) * 64
    feat16 = jnp.concatenate(
        [features,
         jnp.ones((BATCH, 1), jnp.float32),
         jnp.zeros((BATCH, 2), jnp.float32)], axis=1).reshape(NW, 64, 128)
    params = jnp.concatenate(
        [fc_w[0], fc_b, jnp.zeros((2,), jnp.float32)]).astype(jnp.float32)
    wbc = jnp.tile(params[:, None], (1, 16)).reshape(10, 128)
    utab2 = _detranspose(user_table.T)
    mtab2 = _detranspose(movie_table.T)
    return _run(uprow, mprow, upar, mpar, feat16, utab2, mtab2, wbc)


# speedup vs baseline: 3.4183x; 1.0009x over previous
import jax
import jax.numpy as jnp
from jax import lax
from jax.experimental import pallas as pl
from jax.experimental.pallas import tpu as pltpu
from jax.experimental.pallas import tpu_sc as plsc

BATCH = 16384
EMBED_DIM = 64
NUM_FEATURES = 13
NFB = NUM_FEATURES + 1
NC = 2
NS = 16
NW = NC * NS
CHUNK = BATCH // NW
IDX_SUB = 64
NSUB = CHUNK // IDX_SUB
PASSES = 2
SPP = NSUB // PASSES
EPP = CHUNK // PASSES
GPS = IDX_SUB // 16


def _sc_body(uprow_hbm, mprow_hbm, upar_hbm, mpar_hbm, feat_hbm,
             utab_hbm, mtab_hbm, wbc_hbm, out_hbm,
             uprow_v, mprow_v, upar_v, mpar_v, feat_v, wbc_v,
             upair_v, mpair_v, out_v, sem_u, sem_m):
    cid = lax.axis_index("c")
    sid = lax.axis_index("s")
    wid = sid * NC + cid
    base = wid * CHUNK

    pltpu.sync_copy(uprow_hbm.at[wid], uprow_v)
    pltpu.sync_copy(mprow_hbm.at[wid], mprow_v)
    pltpu.sync_copy(upar_hbm.at[pl.ds(base, CHUNK)], upar_v)
    pltpu.sync_copy(mpar_hbm.at[pl.ds(base, CHUNK)], mpar_v)
    pltpu.sync_copy(feat_hbm.at[wid], feat_v)
    pltpu.sync_copy(wbc_hbm, wbc_v)

    lane = lax.iota(jnp.int32, 16)

    def make_group(p):
        def group(g, carry):
            eloc = p * EPP + g * 16
            rloc = g * 16 + lane
            cu = upar_v[pl.ds(eloc, 16)]
            cm = mpar_v[pl.ds(eloc, 16)]
            accs = [jnp.zeros((16,), jnp.float32) for _ in range(4)]
            for d in range(EMBED_DIM):
                u = plsc.load_gather(upair_v, [rloc, cu])
                m = plsc.load_gather(mpair_v, [rloc, cm])
                wv = wbc_v[d // 8, pl.ds((d % 8) * 16, 16)]
                accs[d % 4] = accs[d % 4] + (u * m) * wv
                cu = cu + 1
                cm = cm + 1
            frows = (eloc + lane) >> 3
            cf = ((eloc + lane) & 7) << 4
            for f in range(NFB):
                fv = plsc.load_gather(feat_v, [frows, cf])
                j = EMBED_DIM + f
                accs[f % 4] = accs[f % 4] + fv * wbc_v[
                    j // 8, pl.ds((j % 8) * 16, 16)]
                cf = cf + 1
            out_v[pl.ds(eloc, 16)] = (accs[0] + accs[1]) + (accs[2] + accs[3])
            return carry
        return group

    for p in range(PASSES):
        copies = []
        for j in range(SPP):
            s = p * SPP + j
            copies.append(pltpu.async_copy(
                utab_hbm.at[uprow_v.at[s]],
                upair_v.at[pl.ds(j * IDX_SUB, IDX_SUB)], sem_u))
            copies.append(pltpu.async_copy(
                mtab_hbm.at[mprow_v.at[s]],
                mpair_v.at[pl.ds(j * IDX_SUB, IDX_SUB)], sem_m))
        group = make_group(p)
        for j in range(SPP):
            copies[2 * j].wait()
            copies[2 * j + 1].wait()
            lax.fori_loop(j * GPS, (j + 1) * GPS, group, None)
    pltpu.sync_copy(out_v, out_hbm.at[pl.ds(base, CHUNK)])


TB = 8192
NROW = 1000000
NBLK = (NROW + TB - 1) // TB
PAIR_ROWS = NBLK * (TB // 2)


def _transpose_block(xt_ref, eye_ref, out_ref):
    x = jnp.transpose(xt_ref[...], (1, 0))
    out_ref[:, 0:EMBED_DIM] = x[0:TB // 2, :]
    out_ref[:, EMBED_DIM:128] = x[TB // 2:TB, :]


def _detranspose(tabT):
    eye = jnp.eye(EMBED_DIM, dtype=jnp.float32)
    return pl.pallas_call(
        _transpose_block,
        grid=(NBLK,),
        in_specs=[pl.BlockSpec((EMBED_DIM, TB), lambda i: (0, i)),
                  pl.BlockSpec((EMBED_DIM, EMBED_DIM), lambda i: (0, 0))],
        out_specs=pl.BlockSpec((TB // 2, 128), lambda i: (i, 0)),
        out_shape=jax.ShapeDtypeStruct((PAIR_ROWS, 128), jnp.float32),
    )(tabT, eye)


@jax.jit
def _run(uprow, mprow, upar, mpar, feat16, utab2, mtab2, wbc):
    mesh = plsc.VectorSubcoreMesh(core_axis_name="c", subcore_axis_name="s",
                                  num_cores=NC, num_subcores=NS)
    f = pl.kernel(
        _sc_body,
        out_type=jax.ShapeDtypeStruct((BATCH,), jnp.float32),
        mesh=mesh,
        compiler_params=pltpu.CompilerParams(needs_layout_passes=False),
        scratch_types=[
            pltpu.VMEM((NSUB, IDX_SUB), jnp.int32),
            pltpu.VMEM((NSUB, IDX_SUB), jnp.int32),
            pltpu.VMEM((CHUNK,), jnp.int32),
            pltpu.VMEM((CHUNK,), jnp.int32),
            pltpu.VMEM((CHUNK // 8, 128), jnp.float32),
            pltpu.VMEM((10, 128), jnp.float32),
            pltpu.VMEM((EPP, 128), jnp.float32),
            pltpu.VMEM((EPP, 128), jnp.float32),
            pltpu.VMEM((CHUNK,), jnp.float32),
            pltpu.SemaphoreType.DMA,
            pltpu.SemaphoreType.DMA,
        ],
    )
    return f(uprow, mprow, upar, mpar, feat16, utab2, mtab2, wbc)


def kernel(user_idx, movie_idx, features, user_table, movie_table, fc_w, fc_b):
    ui = user_idx.astype(jnp.int32)
    mi = movie_idx.astype(jnp.int32)
    hb = TB // 2
    uprow = ((ui // TB) * hb + ui % hb).reshape(NW, NSUB, IDX_SUB)
    mprow = ((mi // TB) * hb + mi % hb).reshape(NW, NSUB, IDX_SUB)
    upar = ((ui % TB) // hb) * 64
    mpar = ((mi % TB) // hb) * 64
    feat16 = jnp.concatenate(
        [features,
         jnp.ones((BATCH, 1), jnp.float32),
         jnp.zeros((BATCH, 2), jnp.float32)], axis=1).reshape(NW, 64, 128)
    params = jnp.concatenate(
        [fc_w[0], fc_b, jnp.zeros((2,), jnp.float32)]).astype(jnp.float32)
    wbc = jnp.tile(params[:, None], (1, 16)).reshape(10, 128)
    utab2 = _detranspose(user_table.T)
    mtab2 = _detranspose(movie_table.T)
    return _run(uprow, mprow, upar, mpar, feat16, utab2, mtab2, wbc)


# speedup vs baseline: 3.8271x; 1.1196x over previous
import jax
import jax.numpy as jnp
from jax import lax
from jax.experimental import pallas as pl
from jax.experimental.pallas import tpu as pltpu
from jax.experimental.pallas import tpu_sc as plsc

BATCH = 16384
EMBED_DIM = 64
NUM_FEATURES = 13
NFB = NUM_FEATURES + 1
NC = 2
NS = 16
NW = NC * NS
CHUNK = BATCH // NW
IDX_SUB = 64
NSUB = CHUNK // IDX_SUB
PASSES = 2
SPP = NSUB // PASSES
EPP = CHUNK // PASSES
GPS = IDX_SUB // 16


def _sc_body(uprow_hbm, mprow_hbm, upar_hbm, mpar_hbm, feat_hbm,
             utab_hbm, mtab_hbm, wbc_hbm, out_hbm,
             uprow_v, mprow_v, upar_v, mpar_v, feat_v, wbc_v,
             upair_v, mpair_v, out_v, sem_u, sem_m):
    cid = lax.axis_index("c")
    sid = lax.axis_index("s")
    wid = sid * NC + cid
    base = wid * CHUNK

    pltpu.sync_copy(uprow_hbm.at[wid], uprow_v)
    pltpu.sync_copy(mprow_hbm.at[wid], mprow_v)
    pltpu.sync_copy(upar_hbm.at[pl.ds(base, CHUNK)], upar_v)
    pltpu.sync_copy(mpar_hbm.at[pl.ds(base, CHUNK)], mpar_v)
    pltpu.sync_copy(feat_hbm.at[wid], feat_v)
    pltpu.sync_copy(wbc_hbm, wbc_v)

    lane = lax.iota(jnp.int32, 16)

    def make_group(p):
        def group(g, carry):
            eloc = p * EPP + g * 16
            rloc = g * 16 + lane
            cu = upar_v[pl.ds(eloc, 16)]
            cm = mpar_v[pl.ds(eloc, 16)]
            accs = [jnp.zeros((16,), jnp.float32) for _ in range(4)]
            for d in range(EMBED_DIM):
                u = plsc.load_gather(upair_v, [rloc, cu])
                m = plsc.load_gather(mpair_v, [rloc, cm])
                wv = wbc_v[d // 8, pl.ds((d % 8) * 16, 16)]
                accs[d % 4] = accs[d % 4] + (u * m) * wv
                cu = cu + 1
                cm = cm + 1
            frows = (eloc + lane) >> 3
            cf = ((eloc + lane) & 7) << 4
            for f in range(NFB):
                fv = plsc.load_gather(feat_v, [frows, cf])
                j = EMBED_DIM + f
                accs[f % 4] = accs[f % 4] + fv * wbc_v[
                    j // 8, pl.ds((j % 8) * 16, 16)]
                cf = cf + 1
            out_v[pl.ds(eloc, 16)] = (accs[0] + accs[1]) + (accs[2] + accs[3])
            return carry
        return group

    for p in range(PASSES):
        copies = []
        for j in range(SPP):
            s = p * SPP + j
            copies.append(pltpu.async_copy(
                utab_hbm.at[uprow_v.at[s]],
                upair_v.at[pl.ds(j * IDX_SUB, IDX_SUB)], sem_u))
            copies.append(pltpu.async_copy(
                mtab_hbm.at[mprow_v.at[s]],
                mpair_v.at[pl.ds(j * IDX_SUB, IDX_SUB)], sem_m))
        group = make_group(p)
        for j in range(SPP):
            copies[2 * j].wait()
            copies[2 * j + 1].wait()
            lax.fori_loop(j * GPS, (j + 1) * GPS, group, None)
    pltpu.sync_copy(out_v, out_hbm.at[pl.ds(base, CHUNK)])


TB = 16384
NROW = 1000000
NBLK = (NROW + TB - 1) // TB
PAIR_ROWS = NBLK * (TB // 2)


def _transpose_block(xt_ref, eye_ref, out_ref):
    x = jnp.transpose(xt_ref[...], (1, 0))
    out_ref[:, 0:EMBED_DIM] = x[0:TB // 2, :]
    out_ref[:, EMBED_DIM:128] = x[TB // 2:TB, :]


def _detranspose(tabT):
    eye = jnp.eye(EMBED_DIM, dtype=jnp.float32)
    return pl.pallas_call(
        _transpose_block,
        grid=(NBLK,),
        in_specs=[pl.BlockSpec((EMBED_DIM, TB), lambda i: (0, i)),
                  pl.BlockSpec((EMBED_DIM, EMBED_DIM), lambda i: (0, 0))],
        out_specs=pl.BlockSpec((TB // 2, 128), lambda i: (i, 0)),
        out_shape=jax.ShapeDtypeStruct((PAIR_ROWS, 128), jnp.float32),
    )(tabT, eye)


@jax.jit
def _run(uprow, mprow, upar, mpar, feat16, utab2, mtab2, wbc):
    mesh = plsc.VectorSubcoreMesh(core_axis_name="c", subcore_axis_name="s",
                                  num_cores=NC, num_subcores=NS)
    f = pl.kernel(
        _sc_body,
        out_type=jax.ShapeDtypeStruct((BATCH,), jnp.float32),
        mesh=mesh,
        compiler_params=pltpu.CompilerParams(needs_layout_passes=False),
        scratch_types=[
            pltpu.VMEM((NSUB, IDX_SUB), jnp.int32),
            pltpu.VMEM((NSUB, IDX_SUB), jnp.int32),
            pltpu.VMEM((CHUNK,), jnp.int32),
            pltpu.VMEM((CHUNK,), jnp.int32),
            pltpu.VMEM((CHUNK // 8, 128), jnp.float32),
            pltpu.VMEM((10, 128), jnp.float32),
            pltpu.VMEM((EPP, 128), jnp.float32),
            pltpu.VMEM((EPP, 128), jnp.float32),
            pltpu.VMEM((CHUNK,), jnp.float32),
            pltpu.SemaphoreType.DMA,
            pltpu.SemaphoreType.DMA,
        ],
    )
    return f(uprow, mprow, upar, mpar, feat16, utab2, mtab2, wbc)


def kernel(user_idx, movie_idx, features, user_table, movie_table, fc_w, fc_b):
    ui = user_idx.astype(jnp.int32)
    mi = movie_idx.astype(jnp.int32)
    hb = TB // 2
    uprow = ((ui // TB) * hb + ui % hb).reshape(NW, NSUB, IDX_SUB)
    mprow = ((mi // TB) * hb + mi % hb).reshape(NW, NSUB, IDX_SUB)
    upar = ((ui % TB) // hb) * 64
    mpar = ((mi % TB) // hb) * 64
    feat16 = jnp.concatenate(
        [features,
         jnp.ones((BATCH, 1), jnp.float32),
         jnp.zeros((BATCH, 2), jnp.float32)], axis=1).reshape(NW, 64, 128)
    params = jnp.concatenate(
        [fc_w[0], fc_b, jnp.zeros((2,), jnp.float32)]).astype(jnp.float32)
    wbc = jnp.tile(params[:, None], (1, 16)).reshape(10, 128)
    utab2 = _detranspose(user_table.T)
    mtab2 = _detranspose(movie_table.T)
    return _run(uprow, mprow, upar, mpar, feat16, utab2, mtab2, wbc)


# speedup vs baseline: 4.0669x; 1.0627x over previous
import jax
import jax.numpy as jnp
from jax import lax
from jax.experimental import pallas as pl
from jax.experimental.pallas import tpu as pltpu
from jax.experimental.pallas import tpu_sc as plsc

BATCH = 16384
EMBED_DIM = 64
NUM_FEATURES = 13
NFB = NUM_FEATURES + 1
NC = 2
NS = 16
NW = NC * NS
CHUNK = BATCH // NW
IDX_SUB = 64
NSUB = CHUNK // IDX_SUB
PASSES = 2
SPP = NSUB // PASSES
EPP = CHUNK // PASSES
GPS = IDX_SUB // 16


def _sc_body(uprow_hbm, mprow_hbm, upar_hbm, mpar_hbm, feat_hbm,
             utab_hbm, mtab_hbm, wbc_hbm, out_hbm,
             uprow_v, mprow_v, upar_v, mpar_v, feat_v, wbc_v,
             upair_v, mpair_v, out_v, sem_u, sem_m):
    cid = lax.axis_index("c")
    sid = lax.axis_index("s")
    wid = sid * NC + cid
    base = wid * CHUNK

    pltpu.sync_copy(uprow_hbm.at[wid], uprow_v)
    pltpu.sync_copy(mprow_hbm.at[wid], mprow_v)
    pltpu.sync_copy(upar_hbm.at[pl.ds(base, CHUNK)], upar_v)
    pltpu.sync_copy(mpar_hbm.at[pl.ds(base, CHUNK)], mpar_v)
    pltpu.sync_copy(feat_hbm.at[wid], feat_v)
    pltpu.sync_copy(wbc_hbm, wbc_v)

    lane = lax.iota(jnp.int32, 16)

    def make_group(p):
        def group(g, carry):
            eloc = p * EPP + g * 16
            rloc = g * 16 + lane
            cu = upar_v[pl.ds(eloc, 16)]
            cm = mpar_v[pl.ds(eloc, 16)]
            accs = [jnp.zeros((16,), jnp.float32) for _ in range(4)]
            for d in range(EMBED_DIM):
                u = plsc.load_gather(upair_v, [rloc, cu])
                m = plsc.load_gather(mpair_v, [rloc, cm])
                wv = wbc_v[d // 8, pl.ds((d % 8) * 16, 16)]
                accs[d % 4] = accs[d % 4] + (u * m) * wv
                cu = cu + 1
                cm = cm + 1
            frows = (eloc + lane) >> 3
            cf = ((eloc + lane) & 7) << 4
            for f in range(NFB):
                fv = plsc.load_gather(feat_v, [frows, cf])
                j = EMBED_DIM + f
                accs[f % 4] = accs[f % 4] + fv * wbc_v[
                    j // 8, pl.ds((j % 8) * 16, 16)]
                cf = cf + 1
            out_v[pl.ds(eloc, 16)] = (accs[0] + accs[1]) + (accs[2] + accs[3])
            return carry
        return group

    for p in range(PASSES):
        copies = []
        for j in range(SPP):
            s = p * SPP + j
            copies.append(pltpu.async_copy(
                utab_hbm.at[uprow_v.at[s]],
                upair_v.at[pl.ds(j * IDX_SUB, IDX_SUB)], sem_u))
            copies.append(pltpu.async_copy(
                mtab_hbm.at[mprow_v.at[s]],
                mpair_v.at[pl.ds(j * IDX_SUB, IDX_SUB)], sem_m))
        group = make_group(p)
        for j in range(SPP):
            copies[2 * j].wait()
            copies[2 * j + 1].wait()
            lax.fori_loop(j * GPS, (j + 1) * GPS, group, None)
    pltpu.sync_copy(out_v, out_hbm.at[pl.ds(base, CHUNK)])


TB = 32768
NROW = 1000000
NBLK = (NROW + TB - 1) // TB
PAIR_ROWS = NBLK * (TB // 2)


def _transpose_block(xt_ref, eye_ref, out_ref):
    x = jnp.transpose(xt_ref[...], (1, 0))
    out_ref[:, 0:EMBED_DIM] = x[0:TB // 2, :]
    out_ref[:, EMBED_DIM:128] = x[TB // 2:TB, :]


def _detranspose(tabT):
    eye = jnp.eye(EMBED_DIM, dtype=jnp.float32)
    return pl.pallas_call(
        _transpose_block,
        grid=(NBLK,),
        in_specs=[pl.BlockSpec((EMBED_DIM, TB), lambda i: (0, i)),
                  pl.BlockSpec((EMBED_DIM, EMBED_DIM), lambda i: (0, 0))],
        out_specs=pl.BlockSpec((TB // 2, 128), lambda i: (i, 0)),
        out_shape=jax.ShapeDtypeStruct((PAIR_ROWS, 128), jnp.float32),
    )(tabT, eye)


@jax.jit
def _run(uprow, mprow, upar, mpar, feat16, utab2, mtab2, wbc):
    mesh = plsc.VectorSubcoreMesh(core_axis_name="c", subcore_axis_name="s",
                                  num_cores=NC, num_subcores=NS)
    f = pl.kernel(
        _sc_body,
        out_type=jax.ShapeDtypeStruct((BATCH,), jnp.float32),
        mesh=mesh,
        compiler_params=pltpu.CompilerParams(needs_layout_passes=False),
        scratch_types=[
            pltpu.VMEM((NSUB, IDX_SUB), jnp.int32),
            pltpu.VMEM((NSUB, IDX_SUB), jnp.int32),
            pltpu.VMEM((CHUNK,), jnp.int32),
            pltpu.VMEM((CHUNK,), jnp.int32),
            pltpu.VMEM((CHUNK // 8, 128), jnp.float32),
            pltpu.VMEM((10, 128), jnp.float32),
            pltpu.VMEM((EPP, 128), jnp.float32),
            pltpu.VMEM((EPP, 128), jnp.float32),
            pltpu.VMEM((CHUNK,), jnp.float32),
            pltpu.SemaphoreType.DMA,
            pltpu.SemaphoreType.DMA,
        ],
    )
    return f(uprow, mprow, upar, mpar, feat16, utab2, mtab2, wbc)


def kernel(user_idx, movie_idx, features, user_table, movie_table, fc_w, fc_b):
    ui = user_idx.astype(jnp.int32)
    mi = movie_idx.astype(jnp.int32)
    hb = TB // 2
    uprow = ((ui // TB) * hb + ui % hb).reshape(NW, NSUB, IDX_SUB)
    mprow = ((mi // TB) * hb + mi % hb).reshape(NW, NSUB, IDX_SUB)
    upar = ((ui % TB) // hb) * 64
    mpar = ((mi % TB) // hb) * 64
    feat16 = jnp.concatenate(
        [features,
         jnp.ones((BATCH, 1), jnp.float32),
         jnp.zeros((BATCH, 2), jnp.float32)], axis=1).reshape(NW, 64, 128)
    params = jnp.concatenate(
        [fc_w[0], fc_b, jnp.zeros((2,), jnp.float32)]).astype(jnp.float32)
    wbc = jnp.tile(params[:, None], (1, 16)).reshape(10, 128)
    utab2 = _detranspose(user_table.T)
    mtab2 = _detranspose(movie_table.T)
    return _run(uprow, mprow, upar, mpar, feat16, utab2, mtab2, wbc)
